# baseline (speedup 1.0000x reference)
import jax
import jax.numpy as jnp
from jax import lax
from jax.experimental import pallas as pl
from jax.experimental.pallas import tpu as pltpu

_B, _A, _S, _C = 64, 3, 64, 11
_N = _B * _A * _S * _S
_NB = 4
_R = _NB * _A * _S


def _loss_body(anch_ref, pred_ref, tgt_ref, out_ref):
    i = pl.program_id(0)

    q = jnp.swapaxes(pred_ref[...], 0, 1)
    t4 = tgt_ref[...].reshape(_NB * _A, 6, _S, _S)

    def tch(c):
        return t4[:, c, :, :].reshape(_R, _S)

    obj_flag = tch(0)
    tx, ty, tw, th, tcls = tch(1), tch(2), tch(3), tch(4), tch(5)
    obj_mask = obj_flag == 1.0

    p0 = q[0]
    px = q[1]
    py = q[2]
    pw = q[3]
    ph = q[4]

    g = jnp.maximum(p0, 0.0) + jnp.log1p(jnp.exp(-jnp.abs(p0)))
    noobj_sum = jnp.sum(jnp.where(obj_mask, 0.0, g))
    n_obj = jnp.sum(obj_flag)

    col = lax.broadcasted_iota(jnp.int32, (_R, _S), 1).astype(jnp.float32)
    rowi = lax.broadcasted_iota(jnp.int32, (_R, _S), 0)
    row = (rowi & (_S - 1)).astype(jnp.float32)
    ra = lax.rem(rowi, _A * _S)
    a1 = ra >= _S
    a2 = ra >= 2 * _S
    aw = jnp.where(a2, anch_ref[2, 0], jnp.where(a1, anch_ref[1, 0],
                                                 anch_ref[0, 0]))
    ah = jnp.where(a2, anch_ref[2, 1], jnp.where(a1, anch_ref[1, 1],
                                                 anch_ref[0, 1]))

    sig_x = jax.nn.sigmoid(px)
    sig_y = jax.nn.sigmoid(py)
    pred_cx = col + sig_x
    pred_cy = row + sig_y
    pred_w = aw * jnp.exp(pw)
    pred_h = ah * jnp.exp(ph)

    ax1 = pred_cx - pred_w * 0.5
    ay1 = pred_cy - pred_h * 0.5
    ax2 = pred_cx + pred_w * 0.5
    ay2 = pred_cy + pred_h * 0.5
    bx1 = tx - tw * 0.5
    by1 = ty - th * 0.5
    bx2 = tx + tw * 0.5
    by2 = ty + th * 0.5
    iw = jnp.maximum(jnp.minimum(ax2, bx2) - jnp.maximum(ax1, bx1), 0.0)
    ih = jnp.maximum(jnp.minimum(ay2, by2) - jnp.maximum(ay1, by1), 0.0)
    inter = iw * ih
    area_a = jnp.abs((ax2 - ax1) * (ay2 - ay1))
    area_b = jnp.abs((bx2 - bx1) * (by2 - by1))
    iou = inter / (area_a + area_b - inter + 1e-6)

    sig_obj = jax.nn.sigmoid(p0)
    obj_sum = jnp.sum(jnp.where(obj_mask, (sig_obj - iou) ** 2, 0.0))

    box_elem = ((sig_x - (tx - col)) ** 2 + (sig_y - (ty - row)) ** 2
                + (pw - jnp.log(1e-16 + tw / aw)) ** 2
                + (ph - jnp.log(1e-16 + th / ah)) ** 2)
    box_sum = jnp.sum(jnp.where(obj_mask, box_elem, 0.0))

    esum = jnp.exp(q[5])
    for c in range(1, _C):
        esum = esum + jnp.exp(q[5 + c])
    lse = jnp.log(esum)
    sel = jnp.zeros((_R, _S), jnp.float32)
    for c in range(_C):
        sel = sel + jnp.where(tcls == float(c), q[5 + c], 0.0)
    class_sum = jnp.sum(jnp.where(obj_mask, lse - sel, 0.0))

    partial = jnp.stack([noobj_sum, n_obj, obj_sum, box_sum,
                         class_sum, 0.0, 0.0, 0.0]).reshape(1, 8)

    @pl.when(i == 0)
    def _():
        out_ref[...] = jnp.zeros_like(out_ref)

    out_ref[...] += partial


def kernel(predictions, target, anchor_sizes):
    pv = predictions.transpose(0, 1, 2, 4, 3).reshape(_B * _A * _S, 16, _S)
    tv = target.transpose(0, 1, 4, 2, 3).reshape(_B * _A * 6, _S, _S)

    partials = pl.pallas_call(
        _loss_body,
        grid=(_B // _NB,),
        in_specs=[
            pl.BlockSpec(memory_space=pltpu.SMEM),
            pl.BlockSpec((_R, 16, _S), lambda i: (i, 0, 0)),
            pl.BlockSpec((_NB * _A * 6, _S, _S), lambda i: (i, 0, 0)),
        ],
        out_specs=pl.BlockSpec((1, 8), lambda i: (0, 0)),
        out_shape=jax.ShapeDtypeStruct((1, 8), jnp.float32),
    )(anchor_sizes, pv, tv)

    noobj_sum, n_obj, obj_sum, box_sum, class_sum = (
        partials[0, 0], partials[0, 1], partials[0, 2], partials[0, 3],
        partials[0, 4])
    no_object_loss = noobj_sum / (float(_N) - n_obj)
    object_loss = obj_sum / n_obj
    box_loss = box_sum / (n_obj * 4.0)
    class_loss = class_sum / n_obj
    return 10.0 * box_loss + object_loss + 10.0 * no_object_loss + class_loss


# speedup vs baseline: 1.0548x; 1.0548x over previous
import jax
import jax.numpy as jnp
from jax import lax
from jax.experimental import pallas as pl
from jax.experimental.pallas import tpu as pltpu

_B, _A, _S, _C = 64, 3, 64, 11
_N = _B * _A * _S * _S
_NB = 2
_R = _NB * _A * _S


def _loss_body(anch_ref, pred_ref, tgt_ref, out_ref):
    i = pl.program_id(0)

    q = jnp.swapaxes(pred_ref[...], 0, 1)
    t4 = tgt_ref[...].reshape(_NB * _A, 6, _S, _S)

    def tch(c):
        return t4[:, c, :, :].reshape(_R, _S)

    obj_flag = tch(0)
    tx, ty, tw, th, tcls = tch(1), tch(2), tch(3), tch(4), tch(5)
    obj_mask = obj_flag == 1.0

    p0 = q[0]
    px = q[1]
    py = q[2]
    pw = q[3]
    ph = q[4]

    g = jnp.maximum(p0, 0.0) + jnp.log1p(jnp.exp(-jnp.abs(p0)))
    noobj_sum = jnp.sum(jnp.where(obj_mask, 0.0, g))
    n_obj = jnp.sum(obj_flag)

    col = lax.broadcasted_iota(jnp.int32, (_R, _S), 1).astype(jnp.float32)
    rowi = lax.broadcasted_iota(jnp.int32, (_R, _S), 0)
    row = (rowi & (_S - 1)).astype(jnp.float32)
    ra = lax.rem(rowi, _A * _S)
    a1 = ra >= _S
    a2 = ra >= 2 * _S
    aw = jnp.where(a2, anch_ref[2, 0], jnp.where(a1, anch_ref[1, 0],
                                                 anch_ref[0, 0]))
    ah = jnp.where(a2, anch_ref[2, 1], jnp.where(a1, anch_ref[1, 1],
                                                 anch_ref[0, 1]))

    sig_x = jax.nn.sigmoid(px)
    sig_y = jax.nn.sigmoid(py)
    pred_cx = col + sig_x
    pred_cy = row + sig_y
    pred_w = aw * jnp.exp(pw)
    pred_h = ah * jnp.exp(ph)

    ax1 = pred_cx - pred_w * 0.5
    ay1 = pred_cy - pred_h * 0.5
    ax2 = pred_cx + pred_w * 0.5
    ay2 = pred_cy + pred_h * 0.5
    bx1 = tx - tw * 0.5
    by1 = ty - th * 0.5
    bx2 = tx + tw * 0.5
    by2 = ty + th * 0.5
    iw = jnp.maximum(jnp.minimum(ax2, bx2) - jnp.maximum(ax1, bx1), 0.0)
    ih = jnp.maximum(jnp.minimum(ay2, by2) - jnp.maximum(ay1, by1), 0.0)
    inter = iw * ih
    area_a = jnp.abs((ax2 - ax1) * (ay2 - ay1))
    area_b = jnp.abs((bx2 - bx1) * (by2 - by1))
    iou = inter / (area_a + area_b - inter + 1e-6)

    sig_obj = jax.nn.sigmoid(p0)
    obj_sum = jnp.sum(jnp.where(obj_mask, (sig_obj - iou) ** 2, 0.0))

    box_elem = ((sig_x - (tx - col)) ** 2 + (sig_y - (ty - row)) ** 2
                + (pw - jnp.log(1e-16 + tw / aw)) ** 2
                + (ph - jnp.log(1e-16 + th / ah)) ** 2)
    box_sum = jnp.sum(jnp.where(obj_mask, box_elem, 0.0))

    esum = jnp.exp(q[5])
    for c in range(1, _C):
        esum = esum + jnp.exp(q[5 + c])
    lse = jnp.log(esum)
    sel = jnp.zeros((_R, _S), jnp.float32)
    for c in range(_C):
        sel = sel + jnp.where(tcls == float(c), q[5 + c], 0.0)
    class_sum = jnp.sum(jnp.where(obj_mask, lse - sel, 0.0))

    partial = jnp.stack([noobj_sum, n_obj, obj_sum, box_sum,
                         class_sum, 0.0, 0.0, 0.0]).reshape(1, 8)

    @pl.when(i == 0)
    def _():
        out_ref[...] = jnp.zeros_like(out_ref)

    out_ref[...] += partial


def kernel(predictions, target, anchor_sizes):
    pv = predictions.transpose(0, 1, 2, 4, 3).reshape(_B * _A * _S, 16, _S)
    tv = target.transpose(0, 1, 4, 2, 3).reshape(_B * _A * 6, _S, _S)

    partials = pl.pallas_call(
        _loss_body,
        grid=(_B // _NB,),
        in_specs=[
            pl.BlockSpec(memory_space=pltpu.SMEM),
            pl.BlockSpec((_R, 16, _S), lambda i: (i, 0, 0)),
            pl.BlockSpec((_NB * _A * 6, _S, _S), lambda i: (i, 0, 0)),
        ],
        out_specs=pl.BlockSpec((1, 8), lambda i: (0, 0)),
        out_shape=jax.ShapeDtypeStruct((1, 8), jnp.float32),
    )(anchor_sizes, pv, tv)

    noobj_sum, n_obj, obj_sum, box_sum, class_sum = (
        partials[0, 0], partials[0, 1], partials[0, 2], partials[0, 3],
        partials[0, 4])
    no_object_loss = noobj_sum / (float(_N) - n_obj)
    object_loss = obj_sum / n_obj
    box_loss = box_sum / (n_obj * 4.0)
    class_loss = class_sum / n_obj
    return 10.0 * box_loss + object_loss + 10.0 * no_object_loss + class_loss
